# Initial kernel scaffold; baseline (speedup 1.0000x reference)
import jax
import jax.numpy as jnp
from jax.experimental import pallas as pl


def kernel(h, e, edge_index, A_w, A_b, B_w, B_b, C_w, C_b, D_w, D_b, E_w, E_b, bn_h_gamma, bn_h_beta, bn_e_gamma, bn_e_beta):
    raise NotImplementedError("write your pallas kernel here")


# speedup vs baseline: 3.1688x; 3.1688x over previous
import functools

import jax
import jax.numpy as jnp
from jax import lax
from jax.experimental import pallas as pl
from jax.experimental.pallas import tpu as pltpu
from jax.experimental.pallas import tpu_sc as plsc

NC = 2
NS = 16
NW = NC * NS
L = 16
CHUNK = 128


def _node_mm_body(h, aw, ab, bw, bb, dw, db, ew, eb, ah_o, bh_o, dh_o, eh_o):
    x = h[...]
    ah_o[...] = jnp.dot(x, aw[...], preferred_element_type=jnp.float32) + ab[...]
    bh_o[...] = jnp.dot(x, bw[...], preferred_element_type=jnp.float32) + bb[...]
    dh_o[...] = jnp.dot(x, dw[...], preferred_element_type=jnp.float32) + db[...]
    eh_o[...] = jnp.dot(x, ew[...], preferred_element_type=jnp.float32) + eb[...]


def _edge_mm_body(e, cw, cb, o):
    o[...] = jnp.dot(e[...], cw[...], preferred_element_type=jnp.float32) + cb[...]


def _stage_b_body(psum, bh, bns, bnq, gamma, beta, eee_o, esc_o, esh_o, *,
                  n_edges):
    n = bh.shape[0]
    ss = psum[:n, :] + psum[n:, :]
    eee_o[...] = bh[...] / (ss + 1e-6)
    sums = jnp.sum(bns[...], axis=0, keepdims=True)
    sumsq = jnp.sum(bnq[...], axis=0, keepdims=True)
    mean = sums / n_edges
    var = sumsq / n_edges - mean * mean
    scale = gamma[...] * lax.rsqrt(var + 1e-5)
    shift = beta[...] - mean * scale
    esc_o[...] = jnp.broadcast_to(scale, esc_o.shape)
    esh_o[...] = jnp.broadcast_to(shift, esh_o.shape)


def _stage_c_body(ah, psumh, h, gamma, beta, out):
    n = ah.shape[0]
    s = ah[...] + psumh[:n, :] + psumh[n:, :]
    mean = jnp.mean(s, axis=0, keepdims=True)
    d = s - mean
    var = jnp.mean(d * d, axis=0, keepdims=True)
    y = gamma[...] * d * lax.rsqrt(var + 1e-5) + beta[...]
    out[...] = h[...] + jnp.maximum(y, 0.0)


def _zero_fill(buf, rows):
    d = buf.shape[1]
    zero = jnp.zeros((L,), jnp.float32)

    def body(r, _):
        for g in range(d // L):
            buf[r, pl.ds(g * L, L)] = zero
        return 0

    lax.fori_loop(0, rows, body, 0, unroll=False)


def _spmem_zero(acc, zbuf, row0, rows):
    off = 0
    while off < rows:
        sz = min(CHUNK, rows - off)
        pltpu.sync_copy(zbuf.at[pl.ds(0, sz)],
                        acc.at[pl.ds(pl.multiple_of(row0 + off, 8), sz)])
        off += sz


def _spmem_dump(acc, stage, out_hbm, out_base, row0, rows):
    off = 0
    while off < rows:
        sz = min(CHUNK, rows - off)
        pltpu.sync_copy(acc.at[pl.ds(pl.multiple_of(row0 + off, 8), sz)],
                        stage.at[pl.ds(0, sz)])
        pltpu.sync_copy(stage.at[pl.ds(0, sz)],
                        out_hbm.at[pl.ds(
                            pl.multiple_of(out_base + row0 + off, 8), sz)])
        off += sz


def _row_split(n):
    rps = -(-n // NS)
    rps = -(-rps // 8) * 8
    last = n - rps * (NS - 1)
    assert 0 < last <= rps and rps % 8 == 0
    return rps, last


def _sigmoid(x):
    return 1.0 / (1.0 + jnp.exp(-x))


def _make_pass1(n, e_cnt, d):
    nch = e_cnt // CHUNK
    base_chunks = nch // NW
    extra = nch % NW
    rps, rps_last = _row_split(n)
    grp = d // L
    mesh = plsc.VectorSubcoreMesh(core_axis_name="c", subcore_axis_name="s",
                                  num_cores=NC, num_subcores=NS)

    @functools.partial(
        pl.kernel,
        out_type=[
            jax.ShapeDtypeStruct((e_cnt, d), jnp.float32),
            jax.ShapeDtypeStruct((NC * n, d), jnp.float32),
            jax.ShapeDtypeStruct((NW * d,), jnp.float32),
            jax.ShapeDtypeStruct((NW * d,), jnp.float32),
        ],
        mesh=mesh,
        scratch_types=[
            pltpu.VMEM_SHARED((n, d), jnp.float32),
            pltpu.VMEM((CHUNK,), jnp.int32),
            pltpu.VMEM((CHUNK,), jnp.int32),
            pltpu.VMEM((CHUNK, d), jnp.float32),
            pltpu.VMEM((CHUNK, d), jnp.float32),
            pltpu.VMEM((CHUNK, d), jnp.float32),
            pltpu.VMEM((d,), jnp.float32),
            pltpu.VMEM((d,), jnp.float32),
            pltpu.SemaphoreType.DMA,
            pltpu.SemaphoreType.DMA,
            pltpu.SemaphoreType.DMA,
        ],
    )
    def pass1(ce, dh, eh, src, dst, enew_o, psum_o, bns_o, bnq_o,
              acc, idx_s, idx_d, ce_b, dh_b, eh_b, bns_b, bnq_b,
              sem0, sem1, sem2):
        c = lax.axis_index("c")
        s = lax.axis_index("s")
        wid = s * NC + c

        _zero_fill(dh_b, CHUNK)
        row0 = s * rps

        @pl.when(s < NS - 1)
        def _():
            _spmem_zero(acc, dh_b, row0, rps)

        @pl.when(s == NS - 1)
        def _():
            _spmem_zero(acc, dh_b, row0, rps_last)

        plsc.subcore_barrier()

        carry0 = tuple(jnp.zeros((L,), jnp.float32) for _ in range(2 * grp))
        n_my = base_chunks + jnp.where(wid < extra, 1, 0)

        def chunk_body(k, bn):
            cid = k * NW + wid
            base = cid * CHUNK
            pltpu.sync_copy(src.at[pl.ds(base, CHUNK)], idx_s)
            pltpu.sync_copy(dst.at[pl.ds(base, CHUNK)], idx_d)
            cp0 = pltpu.async_copy(ce.at[pl.ds(base, CHUNK)], ce_b, sem0)
            cp1 = pltpu.async_copy(dh.at[idx_s], dh_b, sem1)
            cp2 = pltpu.async_copy(eh.at[idx_d], eh_b, sem2)
            cp0.wait()
            cp1.wait()
            cp2.wait()

            def row_body(r, bnc):
                out = list(bnc)
                for g in range(grp):
                    sl = pl.ds(g * L, L)
                    x = ce_b[r, sl] + dh_b[r, sl] + eh_b[r, sl]
                    ce_b[r, sl] = x
                    dh_b[r, sl] = _sigmoid(x)
                    out[g] = out[g] + x
                    out[grp + g] = out[grp + g] + x * x
                return tuple(out)

            bn = lax.fori_loop(0, CHUNK, row_body, bn, unroll=False)
            pltpu.sync_copy(ce_b, enew_o.at[pl.ds(base, CHUNK)])
            pltpu.sync_copy(dh_b, acc.at[idx_d], add=True)
            return bn

        bn = lax.fori_loop(0, n_my, chunk_body, carry0, unroll=False)

        for g in range(grp):
            bns_b[pl.ds(g * L, L)] = bn[g]
            bnq_b[pl.ds(g * L, L)] = bn[grp + g]
        pltpu.sync_copy(bns_b, bns_o.at[pl.ds(wid * d, d)])
        pltpu.sync_copy(bnq_b, bnq_o.at[pl.ds(wid * d, d)])

        plsc.subcore_barrier()

        @pl.when(s < NS - 1)
        def _():
            _spmem_dump(acc, dh_b, psum_o, c * n, row0, rps)

        @pl.when(s == NS - 1)
        def _():
            _spmem_dump(acc, dh_b, psum_o, c * n, row0, rps_last)

    return pass1


def _make_pass2(n, e_cnt, d):
    nch = e_cnt // CHUNK
    base_chunks = nch // NW
    extra = nch % NW
    rps, rps_last = _row_split(n)
    grp = d // L
    mesh = plsc.VectorSubcoreMesh(core_axis_name="c", subcore_axis_name="s",
                                  num_cores=NC, num_subcores=NS)

    @functools.partial(
        pl.kernel,
        out_type=[
            jax.ShapeDtypeStruct((e_cnt, d), jnp.float32),
            jax.ShapeDtypeStruct((NC * n, d), jnp.float32),
        ],
        mesh=mesh,
        scratch_types=[
            pltpu.VMEM_SHARED((n, d), jnp.float32),
            pltpu.VMEM((CHUNK,), jnp.int32),
            pltpu.VMEM((CHUNK,), jnp.int32),
            pltpu.VMEM((CHUNK, d), jnp.float32),
            pltpu.VMEM((CHUNK, d), jnp.float32),
            pltpu.VMEM((CHUNK, d), jnp.float32),
            pltpu.VMEM((d,), jnp.float32),
            pltpu.VMEM((d,), jnp.float32),
            pltpu.SemaphoreType.DMA,
            pltpu.SemaphoreType.DMA,
            pltpu.SemaphoreType.DMA,
        ],
    )
    def pass2(enew, ein, eee, src, dst, esc, esh, eout_o, psum_o,
              acc, idx_s, idx_d, en_b, ei_b, g_b, sc_b, sh_b,
              sem0, sem1, sem2):
        c = lax.axis_index("c")
        s = lax.axis_index("s")
        wid = s * NC + c

        _zero_fill(g_b, CHUNK)
        row0 = s * rps

        @pl.when(s < NS - 1)
        def _():
            _spmem_zero(acc, g_b, row0, rps)

        @pl.when(s == NS - 1)
        def _():
            _spmem_zero(acc, g_b, row0, rps_last)

        pltpu.sync_copy(esc, sc_b)
        pltpu.sync_copy(esh, sh_b)
        scv = [sc_b[pl.ds(g * L, L)] for g in range(grp)]
        shv = [sh_b[pl.ds(g * L, L)] for g in range(grp)]

        plsc.subcore_barrier()

        n_my = base_chunks + jnp.where(wid < extra, 1, 0)

        def chunk_body(k, _):
            cid = k * NW + wid
            base = cid * CHUNK
            pltpu.sync_copy(src.at[pl.ds(base, CHUNK)], idx_s)
            pltpu.sync_copy(dst.at[pl.ds(base, CHUNK)], idx_d)
            cp0 = pltpu.async_copy(enew.at[pl.ds(base, CHUNK)], en_b, sem0)
            cp1 = pltpu.async_copy(ein.at[pl.ds(base, CHUNK)], ei_b, sem1)
            cp2 = pltpu.async_copy(eee.at[idx_s], g_b, sem2)
            cp0.wait()
            cp1.wait()
            cp2.wait()

            def row_body(r, carry):
                for g in range(grp):
                    sl = pl.ds(g * L, L)
                    x = en_b[r, sl]
                    sg = _sigmoid(x)
                    g_b[r, sl] = g_b[r, sl] * sg
                    y = jnp.maximum(x * scv[g] + shv[g], 0.0)
                    en_b[r, sl] = ei_b[r, sl] + y
                return carry

            lax.fori_loop(0, CHUNK, row_body, 0, unroll=False)
            pltpu.sync_copy(en_b, eout_o.at[pl.ds(base, CHUNK)])
            pltpu.sync_copy(g_b, acc.at[idx_d], add=True)
            return 0

        lax.fori_loop(0, n_my, chunk_body, 0, unroll=False)

        plsc.subcore_barrier()

        @pl.when(s < NS - 1)
        def _():
            _spmem_dump(acc, g_b, psum_o, c * n, row0, rps)

        @pl.when(s == NS - 1)
        def _():
            _spmem_dump(acc, g_b, psum_o, c * n, row0, rps_last)

    return pass2


def kernel(h, e, edge_index, A_w, A_b, B_w, B_b, C_w, C_b, D_w, D_b, E_w, E_b,
           bn_h_gamma, bn_h_beta, bn_e_gamma, bn_e_beta):
    n, d = h.shape
    e_cnt = e.shape[0]
    src = edge_index[0].astype(jnp.int32)
    dst = edge_index[1].astype(jnp.int32)

    ah, bh, dh, eh = pl.pallas_call(
        _node_mm_body,
        out_shape=[jax.ShapeDtypeStruct((n, d), jnp.float32)] * 4,
    )(h, A_w, A_b.reshape(1, d), B_w, B_b.reshape(1, d),
      D_w, D_b.reshape(1, d), E_w, E_b.reshape(1, d))

    br = 4000
    ce = pl.pallas_call(
        _edge_mm_body,
        grid=(e_cnt // br,),
        in_specs=[
            pl.BlockSpec((br, d), lambda i: (i, 0)),
            pl.BlockSpec((d, d), lambda i: (0, 0)),
            pl.BlockSpec((1, d), lambda i: (0, 0)),
        ],
        out_specs=pl.BlockSpec((br, d), lambda i: (i, 0)),
        out_shape=jax.ShapeDtypeStruct((e_cnt, d), jnp.float32),
    )(e, C_w, C_b.reshape(1, d))

    enew, psum, bns, bnq = _make_pass1(n, e_cnt, d)(ce, dh, eh, src, dst)

    eee, esc, esh = pl.pallas_call(
        functools.partial(_stage_b_body, n_edges=float(e_cnt)),
        out_shape=[
            jax.ShapeDtypeStruct((n, d), jnp.float32),
            jax.ShapeDtypeStruct((8, d), jnp.float32),
            jax.ShapeDtypeStruct((8, d), jnp.float32),
        ],
    )(psum, bh, bns.reshape(NW, d), bnq.reshape(NW, d),
      bn_e_gamma.reshape(1, d), bn_e_beta.reshape(1, d))

    e_out, psumh = _make_pass2(n, e_cnt, d)(enew, e, eee, src, dst,
                                            esc[0], esh[0])

    h_out = pl.pallas_call(
        _stage_c_body,
        out_shape=jax.ShapeDtypeStruct((n, d), jnp.float32),
    )(ah, psumh, h, bn_h_gamma.reshape(1, d), bn_h_beta.reshape(1, d))

    return (h_out, e_out)


# speedup vs baseline: 5.4186x; 1.7100x over previous
import functools

import jax
import jax.numpy as jnp
from jax import lax
from jax.experimental import pallas as pl
from jax.experimental.pallas import tpu as pltpu
from jax.experimental.pallas import tpu_sc as plsc

NC = 2
NS = 16
NW = NC * NS
L = 16
CHUNK = 64


def _node_mm_body(h, aw, ab, bw, bb, dw, db, ew, eb, ah_o, bh_o, dh_o, eh_o):
    x = h[...]
    ah_o[...] = jnp.dot(x, aw[...], preferred_element_type=jnp.float32) + ab[...]
    bh_o[...] = jnp.dot(x, bw[...], preferred_element_type=jnp.float32) + bb[...]
    dh_o[...] = jnp.dot(x, dw[...], preferred_element_type=jnp.float32) + db[...]
    eh_o[...] = jnp.dot(x, ew[...], preferred_element_type=jnp.float32) + eb[...]


def _edge_mm_body(e, cw, cb, o):
    o[...] = jnp.dot(e[...], cw[...], preferred_element_type=jnp.float32) + cb[...]


def _stage_b_body(psum, bh, bns, bnq, gamma, beta, eee_o, esc_o, esh_o, *,
                  n_edges):
    n = bh.shape[0]
    ss = psum[:n, :] + psum[n:, :]
    eee_o[...] = bh[...] / (ss + 1e-6)
    sums = jnp.sum(bns[...], axis=0, keepdims=True)
    sumsq = jnp.sum(bnq[...], axis=0, keepdims=True)
    mean = sums / n_edges
    var = sumsq / n_edges - mean * mean
    scale = gamma[...] * lax.rsqrt(var + 1e-5)
    shift = beta[...] - mean * scale
    esc_o[...] = jnp.broadcast_to(scale, esc_o.shape)
    esh_o[...] = jnp.broadcast_to(shift, esh_o.shape)


def _stage_c_body(ah, psumh, h, gamma, beta, out):
    n = ah.shape[0]
    s = ah[...] + psumh[:n, :] + psumh[n:, :]
    mean = jnp.mean(s, axis=0, keepdims=True)
    d = s - mean
    var = jnp.mean(d * d, axis=0, keepdims=True)
    y = gamma[...] * d * lax.rsqrt(var + 1e-5) + beta[...]
    out[...] = h[...] + jnp.maximum(y, 0.0)


def _zero_fill(buf, rows):
    d = buf.shape[1]
    zero = jnp.zeros((L,), jnp.float32)

    def body(r, _):
        for g in range(d // L):
            buf[r, pl.ds(g * L, L)] = zero
        return 0

    lax.fori_loop(0, rows, body, 0, unroll=False)


def _spmem_zero(acc, zbuf, row0, rows):
    off = 0
    while off < rows:
        sz = min(CHUNK, rows - off)
        pltpu.sync_copy(zbuf.at[pl.ds(0, sz)],
                        acc.at[pl.ds(pl.multiple_of(row0 + off, 8), sz)])
        off += sz


def _spmem_dump(acc, stage, out_hbm, out_base, row0, rows):
    off = 0
    while off < rows:
        sz = min(CHUNK, rows - off)
        pltpu.sync_copy(acc.at[pl.ds(pl.multiple_of(row0 + off, 8), sz)],
                        stage.at[pl.ds(0, sz)])
        pltpu.sync_copy(stage.at[pl.ds(0, sz)],
                        out_hbm.at[pl.ds(
                            pl.multiple_of(out_base + row0 + off, 8), sz)])
        off += sz


def _row_split(n):
    rps = -(-n // NS)
    rps = -(-rps // 8) * 8
    last = n - rps * (NS - 1)
    assert 0 < last <= rps and rps % 8 == 0
    return rps, last


def _sigmoid(x):
    return 1.0 / (1.0 + jnp.exp(-x))


def _make_pass1(n, e_cnt, d):
    nch = e_cnt // CHUNK
    base_chunks = nch // NW
    extra = nch % NW
    rps, rps_last = _row_split(n)
    grp = d // L
    mesh = plsc.VectorSubcoreMesh(core_axis_name="c", subcore_axis_name="s",
                                  num_cores=NC, num_subcores=NS)

    kmax = base_chunks + (1 if extra else 0) + 1
    if kmax % 2:
        kmax += 1

    @functools.partial(
        pl.kernel,
        out_type=[
            jax.ShapeDtypeStruct((e_cnt, d), jnp.float32),
            jax.ShapeDtypeStruct((NC * n, d), jnp.float32),
            jax.ShapeDtypeStruct((NW * d,), jnp.float32),
            jax.ShapeDtypeStruct((NW * d,), jnp.float32),
        ],
        mesh=mesh,
        scratch_types=[
            pltpu.VMEM_SHARED((n, d), jnp.float32),
            pltpu.VMEM((CHUNK,), jnp.int32),
            pltpu.VMEM((CHUNK,), jnp.int32),
            pltpu.VMEM((CHUNK,), jnp.int32),
            pltpu.VMEM((CHUNK,), jnp.int32),
            pltpu.VMEM((CHUNK,), jnp.int32),
            pltpu.VMEM((CHUNK,), jnp.int32),
            pltpu.VMEM((CHUNK, d), jnp.float32),
            pltpu.VMEM((CHUNK, d), jnp.float32),
            pltpu.VMEM((CHUNK, d), jnp.float32),
            pltpu.VMEM((CHUNK, d), jnp.float32),
            pltpu.VMEM((CHUNK, d), jnp.float32),
            pltpu.VMEM((CHUNK, d), jnp.float32),
            pltpu.VMEM((d,), jnp.float32),
            pltpu.VMEM((d,), jnp.float32),
            pltpu.SemaphoreType.DMA((2,)),
            pltpu.SemaphoreType.DMA((2,)),
            pltpu.SemaphoreType.DMA((2,)),
            pltpu.SemaphoreType.DMA((2,)),
            pltpu.SemaphoreType.DMA((2,)),
            pltpu.SemaphoreType.DMA((2,)),
            pltpu.SemaphoreType.DMA((2,)),
        ],
    )
    def pass1(ce, dh, eh, src, dst, enew_o, psum_o, bns_o, bnq_o,
              acc, is0, is1, id0, id1, ic0, ic1, ce0, ce1, dh0, dh1, eh0, eh1,
              bns_b, bnq_b, sis, sid, sce, sdh, seh, soe, sos):
        c = lax.axis_index("c")
        s = lax.axis_index("s")
        wid = s * NC + c
        isb, idb, icb = (is0, is1), (id0, id1), (ic0, ic1)
        ceb, dhb, ehb = (ce0, ce1), (dh0, dh1), (eh0, eh1)

        _zero_fill(dh0, CHUNK)
        row0 = s * rps

        @pl.when(s < NS - 1)
        def _():
            _spmem_zero(acc, dh0, row0, rps)

        @pl.when(s == NS - 1)
        def _():
            _spmem_zero(acc, dh0, row0, rps_last)

        plsc.subcore_barrier()

        n_my = base_chunks + jnp.where(wid < extra, 1, 0)

        def idx_issue(k, b):
            base = (k * NW + wid) * CHUNK
            pltpu.async_copy(src.at[pl.ds(base, CHUNK)], isb[b], sis.at[b])
            pltpu.async_copy(dst.at[pl.ds(base, CHUNK)], idb[b], sid.at[b])

        def idx_wait(b):
            pltpu.make_async_copy(src.at[pl.ds(0, CHUNK)], isb[b],
                                  sis.at[b]).wait()
            pltpu.make_async_copy(dst.at[pl.ds(0, CHUNK)], idb[b],
                                  sid.at[b]).wait()

        def in_issue(k, b):
            base = (k * NW + wid) * CHUNK
            pltpu.async_copy(ce.at[pl.ds(base, CHUNK)], ceb[b], sce.at[b])
            pltpu.async_copy(dh.at[isb[b]], dhb[b], sdh.at[b])
            pltpu.async_copy(eh.at[idb[b]], ehb[b], seh.at[b])

        def in_wait(b):
            pltpu.make_async_copy(ce.at[pl.ds(0, CHUNK)], ceb[b],
                                  sce.at[b]).wait()
            pltpu.make_async_copy(dh.at[isb[b]], dhb[b], sdh.at[b]).wait()
            pltpu.make_async_copy(eh.at[idb[b]], ehb[b], seh.at[b]).wait()

        def out_issue(k, b):
            base = (k * NW + wid) * CHUNK
            pltpu.async_copy(ceb[b], enew_o.at[pl.ds(base, CHUNK)], soe.at[b])
            pltpu.async_copy(dhb[b], acc.at[icb[b]], sos.at[b], add=True)

        def out_wait(b):
            pltpu.make_async_copy(ceb[b], enew_o.at[pl.ds(0, CHUNK)],
                                  soe.at[b]).wait()
            pltpu.make_async_copy(dhb[b], acc.at[icb[b]], sos.at[b]).wait()

        def idx_keep(b):
            for g in range(CHUNK // L):
                icb[b][pl.ds(g * L, L)] = idb[b][pl.ds(g * L, L)]

        idx_issue(0, 0)
        idx_issue(1, 1)
        idx_wait(0)
        in_issue(0, 0)

        carry0 = tuple(jnp.zeros((L,), jnp.float32) for _ in range(2 * grp))

        def outer_body(kk, bn):
            for b in (0, 1):
                k = 2 * kk + b
                nb = 1 - b
                live = k < n_my

                @pl.when(live)
                def _():
                    in_wait(b)
                    idx_keep(b)

                @pl.when(jnp.logical_and(k >= 1, k - 1 < n_my))
                def _():
                    out_wait(nb)

                @pl.when(k + 1 < n_my)
                def _():
                    idx_wait(nb)
                    in_issue(k + 1, nb)

                @pl.when(k + 2 < n_my)
                def _():
                    idx_issue(k + 2, b)

                def row_body(r, lc):
                    out = list(lc)
                    for g in range(grp):
                        sl = pl.ds(g * L, L)
                        x = ceb[b][r, sl] + dhb[b][r, sl] + ehb[b][r, sl]
                        ceb[b][r, sl] = x
                        dhb[b][r, sl] = _sigmoid(x)
                        out[g] = out[g] + x
                        out[grp + g] = out[grp + g] + x * x
                    return tuple(out)

                local0 = tuple(jnp.zeros((L,), jnp.float32)
                               for _ in range(2 * grp))
                local = lax.fori_loop(0, CHUNK, row_body, local0, unroll=False)
                mask = jnp.where(live, 1.0, 0.0)
                bn = tuple(bn[i] + local[i] * mask for i in range(2 * grp))

                @pl.when(live)
                def _():
                    out_issue(k, b)
            return bn

        bn = lax.fori_loop(0, kmax // 2, outer_body, carry0, unroll=False)

        for g in range(grp):
            bns_b[pl.ds(g * L, L)] = bn[g]
            bnq_b[pl.ds(g * L, L)] = bn[grp + g]
        pltpu.sync_copy(bns_b, bns_o.at[pl.ds(wid * d, d)])
        pltpu.sync_copy(bnq_b, bnq_o.at[pl.ds(wid * d, d)])

        plsc.subcore_barrier()

        @pl.when(s < NS - 1)
        def _():
            _spmem_dump(acc, dh0, psum_o, c * n, row0, rps)

        @pl.when(s == NS - 1)
        def _():
            _spmem_dump(acc, dh0, psum_o, c * n, row0, rps_last)

    return pass1


def _make_pass2(n, e_cnt, d):
    nch = e_cnt // CHUNK
    base_chunks = nch // NW
    extra = nch % NW
    rps, rps_last = _row_split(n)
    grp = d // L
    mesh = plsc.VectorSubcoreMesh(core_axis_name="c", subcore_axis_name="s",
                                  num_cores=NC, num_subcores=NS)

    kmax = base_chunks + (1 if extra else 0) + 1
    if kmax % 2:
        kmax += 1

    @functools.partial(
        pl.kernel,
        out_type=[
            jax.ShapeDtypeStruct((e_cnt, d), jnp.float32),
            jax.ShapeDtypeStruct((NC * n, d), jnp.float32),
        ],
        mesh=mesh,
        scratch_types=[
            pltpu.VMEM_SHARED((n, d), jnp.float32),
            pltpu.VMEM((CHUNK,), jnp.int32),
            pltpu.VMEM((CHUNK,), jnp.int32),
            pltpu.VMEM((CHUNK,), jnp.int32),
            pltpu.VMEM((CHUNK,), jnp.int32),
            pltpu.VMEM((CHUNK,), jnp.int32),
            pltpu.VMEM((CHUNK,), jnp.int32),
            pltpu.VMEM((CHUNK, d), jnp.float32),
            pltpu.VMEM((CHUNK, d), jnp.float32),
            pltpu.VMEM((CHUNK, d), jnp.float32),
            pltpu.VMEM((CHUNK, d), jnp.float32),
            pltpu.VMEM((CHUNK, d), jnp.float32),
            pltpu.VMEM((CHUNK, d), jnp.float32),
            pltpu.VMEM((d,), jnp.float32),
            pltpu.VMEM((d,), jnp.float32),
            pltpu.SemaphoreType.DMA((2,)),
            pltpu.SemaphoreType.DMA((2,)),
            pltpu.SemaphoreType.DMA((2,)),
            pltpu.SemaphoreType.DMA((2,)),
            pltpu.SemaphoreType.DMA((2,)),
            pltpu.SemaphoreType.DMA((2,)),
            pltpu.SemaphoreType.DMA((2,)),
        ],
    )
    def pass2(enew, ein, eee, src, dst, esc, esh, eout_o, psum_o,
              acc, is0, is1, id0, id1, ic0, ic1, en0, en1, ei0, ei1, g0, g1,
              sc_b, sh_b, sis, sid, sen, sei, sg, soe, sos):
        c = lax.axis_index("c")
        s = lax.axis_index("s")
        wid = s * NC + c
        isb, idb, icb = (is0, is1), (id0, id1), (ic0, ic1)
        enb, eib, gb = (en0, en1), (ei0, ei1), (g0, g1)

        _zero_fill(g0, CHUNK)
        row0 = s * rps

        @pl.when(s < NS - 1)
        def _():
            _spmem_zero(acc, g0, row0, rps)

        @pl.when(s == NS - 1)
        def _():
            _spmem_zero(acc, g0, row0, rps_last)

        pltpu.sync_copy(esc, sc_b)
        pltpu.sync_copy(esh, sh_b)
        scv = [sc_b[pl.ds(g * L, L)] for g in range(grp)]
        shv = [sh_b[pl.ds(g * L, L)] for g in range(grp)]

        plsc.subcore_barrier()

        n_my = base_chunks + jnp.where(wid < extra, 1, 0)

        def idx_issue(k, b):
            base = (k * NW + wid) * CHUNK
            pltpu.async_copy(src.at[pl.ds(base, CHUNK)], isb[b], sis.at[b])
            pltpu.async_copy(dst.at[pl.ds(base, CHUNK)], idb[b], sid.at[b])

        def idx_wait(b):
            pltpu.make_async_copy(src.at[pl.ds(0, CHUNK)], isb[b],
                                  sis.at[b]).wait()
            pltpu.make_async_copy(dst.at[pl.ds(0, CHUNK)], idb[b],
                                  sid.at[b]).wait()

        def in_issue(k, b):
            base = (k * NW + wid) * CHUNK
            pltpu.async_copy(enew.at[pl.ds(base, CHUNK)], enb[b], sen.at[b])
            pltpu.async_copy(ein.at[pl.ds(base, CHUNK)], eib[b], sei.at[b])
            pltpu.async_copy(eee.at[isb[b]], gb[b], sg.at[b])

        def in_wait(b):
            pltpu.make_async_copy(enew.at[pl.ds(0, CHUNK)], enb[b],
                                  sen.at[b]).wait()
            pltpu.make_async_copy(ein.at[pl.ds(0, CHUNK)], eib[b],
                                  sei.at[b]).wait()
            pltpu.make_async_copy(eee.at[isb[b]], gb[b], sg.at[b]).wait()

        def out_issue(k, b):
            base = (k * NW + wid) * CHUNK
            pltpu.async_copy(enb[b], eout_o.at[pl.ds(base, CHUNK)], soe.at[b])
            pltpu.async_copy(gb[b], acc.at[icb[b]], sos.at[b], add=True)

        def out_wait(b):
            pltpu.make_async_copy(enb[b], eout_o.at[pl.ds(0, CHUNK)],
                                  soe.at[b]).wait()
            pltpu.make_async_copy(gb[b], acc.at[icb[b]], sos.at[b]).wait()

        def idx_keep(b):
            for g in range(CHUNK // L):
                icb[b][pl.ds(g * L, L)] = idb[b][pl.ds(g * L, L)]

        idx_issue(0, 0)
        idx_issue(1, 1)
        idx_wait(0)
        in_issue(0, 0)

        def outer_body(kk, carry):
            for b in (0, 1):
                k = 2 * kk + b
                nb = 1 - b
                live = k < n_my

                @pl.when(live)
                def _():
                    in_wait(b)
                    idx_keep(b)

                @pl.when(jnp.logical_and(k >= 1, k - 1 < n_my))
                def _():
                    out_wait(nb)

                @pl.when(k + 1 < n_my)
                def _():
                    idx_wait(nb)
                    in_issue(k + 1, nb)

                @pl.when(k + 2 < n_my)
                def _():
                    idx_issue(k + 2, b)

                def row_body(r, rc):
                    for g in range(grp):
                        sl = pl.ds(g * L, L)
                        x = enb[b][r, sl]
                        sg_v = _sigmoid(x)
                        gb[b][r, sl] = gb[b][r, sl] * sg_v
                        y = jnp.maximum(x * scv[g] + shv[g], 0.0)
                        enb[b][r, sl] = eib[b][r, sl] + y
                    return rc

                lax.fori_loop(0, CHUNK, row_body, 0, unroll=False)

                @pl.when(live)
                def _():
                    out_issue(k, b)
            return carry

        lax.fori_loop(0, kmax // 2, outer_body, 0, unroll=False)

        plsc.subcore_barrier()

        @pl.when(s < NS - 1)
        def _():
            _spmem_dump(acc, g0, psum_o, c * n, row0, rps)

        @pl.when(s == NS - 1)
        def _():
            _spmem_dump(acc, g0, psum_o, c * n, row0, rps_last)

    return pass2


def kernel(h, e, edge_index, A_w, A_b, B_w, B_b, C_w, C_b, D_w, D_b, E_w, E_b,
           bn_h_gamma, bn_h_beta, bn_e_gamma, bn_e_beta):
    n, d = h.shape
    e_cnt = e.shape[0]
    src = edge_index[0].astype(jnp.int32)
    dst = edge_index[1].astype(jnp.int32)

    ah, bh, dh, eh = pl.pallas_call(
        _node_mm_body,
        out_shape=[jax.ShapeDtypeStruct((n, d), jnp.float32)] * 4,
    )(h, A_w, A_b.reshape(1, d), B_w, B_b.reshape(1, d),
      D_w, D_b.reshape(1, d), E_w, E_b.reshape(1, d))

    br = 4000
    ce = pl.pallas_call(
        _edge_mm_body,
        grid=(e_cnt // br,),
        in_specs=[
            pl.BlockSpec((br, d), lambda i: (i, 0)),
            pl.BlockSpec((d, d), lambda i: (0, 0)),
            pl.BlockSpec((1, d), lambda i: (0, 0)),
        ],
        out_specs=pl.BlockSpec((br, d), lambda i: (i, 0)),
        out_shape=jax.ShapeDtypeStruct((e_cnt, d), jnp.float32),
    )(e, C_w, C_b.reshape(1, d))

    enew, psum, bns, bnq = _make_pass1(n, e_cnt, d)(ce, dh, eh, src, dst)

    eee, esc, esh = pl.pallas_call(
        functools.partial(_stage_b_body, n_edges=float(e_cnt)),
        out_shape=[
            jax.ShapeDtypeStruct((n, d), jnp.float32),
            jax.ShapeDtypeStruct((8, d), jnp.float32),
            jax.ShapeDtypeStruct((8, d), jnp.float32),
        ],
    )(psum, bh, bns.reshape(NW, d), bnq.reshape(NW, d),
      bn_e_gamma.reshape(1, d), bn_e_beta.reshape(1, d))

    e_out, psumh = _make_pass2(n, e_cnt, d)(enew, e, eee, src, dst,
                                            esc[0], esh[0])

    h_out = pl.pallas_call(
        _stage_c_body,
        out_shape=jax.ShapeDtypeStruct((n, d), jnp.float32),
    )(ah, psumh, h, bn_h_gamma.reshape(1, d), bn_h_beta.reshape(1, d))

    return (h_out, e_out)


# speedup vs baseline: 5.4955x; 1.0142x over previous
import functools

import jax
import jax.numpy as jnp
from jax import lax
from jax.experimental import pallas as pl
from jax.experimental.pallas import tpu as pltpu
from jax.experimental.pallas import tpu_sc as plsc

NC = 2
NS = 16
NW = NC * NS
L = 16
CHUNK = 64


def _node_mm2_body(h, aw, ab, bw, bb, a_o, b_o):
    x = h[...]
    a_o[...] = jnp.dot(x, aw[...], preferred_element_type=jnp.float32) + ab[...]
    b_o[...] = jnp.dot(x, bw[...], preferred_element_type=jnp.float32) + bb[...]


def _edge_mm_body(e, cw, cb, o):
    o[...] = jnp.dot(e[...], cw[...], preferred_element_type=jnp.float32) + cb[...]


def _eout_body(enew, ein, esc, esh, o):
    y = jnp.maximum(enew[...] * esc[...] + esh[...], 0.0)
    o[...] = ein[...] + y


def _stage_b_body(psum, bh, bns, bnq, gamma, beta, eee_o, esc_o, esh_o, *,
                  n_edges):
    n = bh.shape[0]
    ss = psum[:n, :] + psum[n:, :]
    eee_o[...] = bh[...] / (ss + 1e-6)
    sums = jnp.sum(bns[...], axis=0, keepdims=True)
    sumsq = jnp.sum(bnq[...], axis=0, keepdims=True)
    mean = sums / n_edges
    var = sumsq / n_edges - mean * mean
    scale = gamma[...] * lax.rsqrt(var + 1e-5)
    shift = beta[...] - mean * scale
    esc_o[...] = jnp.broadcast_to(scale, esc_o.shape)
    esh_o[...] = jnp.broadcast_to(shift, esh_o.shape)


def _stage_c_body(ah, psumh, h, gamma, beta, out):
    n = ah.shape[0]
    s = ah[...] + psumh[:n, :] + psumh[n:, :]
    mean = jnp.mean(s, axis=0, keepdims=True)
    d = s - mean
    var = jnp.mean(d * d, axis=0, keepdims=True)
    y = gamma[...] * d * lax.rsqrt(var + 1e-5) + beta[...]
    out[...] = h[...] + jnp.maximum(y, 0.0)


def _zero_fill(buf, rows):
    d = buf.shape[1]
    zero = jnp.zeros((L,), jnp.float32)

    def body(r, _):
        for g in range(d // L):
            buf[r, pl.ds(g * L, L)] = zero
        return 0

    lax.fori_loop(0, rows, body, 0, unroll=False)


def _spmem_zero(acc, zbuf, row0, rows):
    off = 0
    while off < rows:
        sz = min(CHUNK, rows - off)
        pltpu.sync_copy(zbuf.at[pl.ds(0, sz)],
                        acc.at[pl.ds(pl.multiple_of(row0 + off, 8), sz)])
        off += sz


def _spmem_dump(acc, stage, out_hbm, out_base, row0, rows):
    off = 0
    while off < rows:
        sz = min(CHUNK, rows - off)
        pltpu.sync_copy(acc.at[pl.ds(pl.multiple_of(row0 + off, 8), sz)],
                        stage.at[pl.ds(0, sz)])
        pltpu.sync_copy(stage.at[pl.ds(0, sz)],
                        out_hbm.at[pl.ds(
                            pl.multiple_of(out_base + row0 + off, 8), sz)])
        off += sz


def _row_split(n):
    rps = -(-n // NS)
    rps = -(-rps // 8) * 8
    last = n - rps * (NS - 1)
    assert 0 < last <= rps and rps % 8 == 0
    return rps, last


def _sigmoid(x):
    return 1.0 / (1.0 + jnp.exp(-x))


def _make_pass1(n, e_cnt, d):
    nch = e_cnt // CHUNK
    base_chunks = nch // NW
    extra = nch % NW
    rps, rps_last = _row_split(n)
    grp = d // L
    mesh = plsc.VectorSubcoreMesh(core_axis_name="c", subcore_axis_name="s",
                                  num_cores=NC, num_subcores=NS)

    kmax = base_chunks + (1 if extra else 0) + 1
    if kmax % 2:
        kmax += 1

    @functools.partial(
        pl.kernel,
        out_type=[
            jax.ShapeDtypeStruct((e_cnt, d), jnp.float32),
            jax.ShapeDtypeStruct((NC * n, d), jnp.float32),
            jax.ShapeDtypeStruct((NW * d,), jnp.float32),
            jax.ShapeDtypeStruct((NW * d,), jnp.float32),
        ],
        mesh=mesh,
        scratch_types=[
            pltpu.VMEM_SHARED((n, d), jnp.float32),
            pltpu.VMEM((CHUNK,), jnp.int32),
            pltpu.VMEM((CHUNK,), jnp.int32),
            pltpu.VMEM((CHUNK,), jnp.int32),
            pltpu.VMEM((CHUNK,), jnp.int32),
            pltpu.VMEM((CHUNK,), jnp.int32),
            pltpu.VMEM((CHUNK,), jnp.int32),
            pltpu.VMEM((CHUNK, d), jnp.float32),
            pltpu.VMEM((CHUNK, d), jnp.float32),
            pltpu.VMEM((CHUNK, d), jnp.float32),
            pltpu.VMEM((CHUNK, d), jnp.float32),
            pltpu.VMEM((CHUNK, d), jnp.float32),
            pltpu.VMEM((CHUNK, d), jnp.float32),
            pltpu.VMEM((d,), jnp.float32),
            pltpu.VMEM((d,), jnp.float32),
            pltpu.SemaphoreType.DMA((2,)),
            pltpu.SemaphoreType.DMA((2,)),
            pltpu.SemaphoreType.DMA((2,)),
            pltpu.SemaphoreType.DMA((2,)),
            pltpu.SemaphoreType.DMA((2,)),
            pltpu.SemaphoreType.DMA((2,)),
            pltpu.SemaphoreType.DMA((2,)),
        ],
    )
    def pass1(ce, dh, eh, src, dst, enew_o, psum_o, bns_o, bnq_o,
              acc, is0, is1, id0, id1, ic0, ic1, ce0, ce1, dh0, dh1, eh0, eh1,
              bns_b, bnq_b, sis, sid, sce, sdh, seh, soe, sos):
        c = lax.axis_index("c")
        s = lax.axis_index("s")
        wid = s * NC + c
        isb, idb, icb = (is0, is1), (id0, id1), (ic0, ic1)
        ceb, dhb, ehb = (ce0, ce1), (dh0, dh1), (eh0, eh1)

        _zero_fill(dh0, CHUNK)
        row0 = s * rps

        @pl.when(s < NS - 1)
        def _():
            _spmem_zero(acc, dh0, row0, rps)

        @pl.when(s == NS - 1)
        def _():
            _spmem_zero(acc, dh0, row0, rps_last)

        plsc.subcore_barrier()

        n_my = base_chunks + jnp.where(wid < extra, 1, 0)

        def idx_issue(k, b):
            base = (k * NW + wid) * CHUNK
            pltpu.async_copy(src.at[pl.ds(base, CHUNK)], isb[b], sis.at[b])
            pltpu.async_copy(dst.at[pl.ds(base, CHUNK)], idb[b], sid.at[b])

        def idx_wait(b):
            pltpu.make_async_copy(src.at[pl.ds(0, CHUNK)], isb[b],
                                  sis.at[b]).wait()
            pltpu.make_async_copy(dst.at[pl.ds(0, CHUNK)], idb[b],
                                  sid.at[b]).wait()

        def in_issue(k, b):
            base = (k * NW + wid) * CHUNK
            pltpu.async_copy(ce.at[pl.ds(base, CHUNK)], ceb[b], sce.at[b])
            pltpu.async_copy(dh.at[isb[b]], dhb[b], sdh.at[b])
            pltpu.async_copy(eh.at[idb[b]], ehb[b], seh.at[b])

        def in_wait(b):
            pltpu.make_async_copy(ce.at[pl.ds(0, CHUNK)], ceb[b],
                                  sce.at[b]).wait()
            pltpu.make_async_copy(dh.at[isb[b]], dhb[b], sdh.at[b]).wait()
            pltpu.make_async_copy(eh.at[idb[b]], ehb[b], seh.at[b]).wait()

        def out_issue(k, b):
            base = (k * NW + wid) * CHUNK
            pltpu.async_copy(ceb[b], enew_o.at[pl.ds(base, CHUNK)], soe.at[b])
            pltpu.async_copy(dhb[b], acc.at[icb[b]], sos.at[b], add=True)

        def out_wait(b):
            pltpu.make_async_copy(ceb[b], enew_o.at[pl.ds(0, CHUNK)],
                                  soe.at[b]).wait()
            pltpu.make_async_copy(dhb[b], acc.at[icb[b]], sos.at[b]).wait()

        def idx_keep(b):
            for g in range(CHUNK // L):
                icb[b][pl.ds(g * L, L)] = idb[b][pl.ds(g * L, L)]

        idx_issue(0, 0)
        idx_issue(1, 1)
        idx_wait(0)
        in_issue(0, 0)

        carry0 = tuple(jnp.zeros((L,), jnp.float32) for _ in range(2 * grp))

        def outer_body(kk, bn):
            for b in (0, 1):
                k = 2 * kk + b
                nb = 1 - b
                live = k < n_my

                @pl.when(live)
                def _():
                    in_wait(b)
                    idx_keep(b)

                @pl.when(jnp.logical_and(k >= 1, k - 1 < n_my))
                def _():
                    out_wait(nb)

                @pl.when(k + 1 < n_my)
                def _():
                    idx_wait(nb)
                    in_issue(k + 1, nb)

                @pl.when(k + 2 < n_my)
                def _():
                    idx_issue(k + 2, b)

                def row_body(r, lc):
                    out = list(lc)
                    for g in range(grp):
                        sl = pl.ds(g * L, L)
                        x = ceb[b][r, sl] + dhb[b][r, sl] + ehb[b][r, sl]
                        ceb[b][r, sl] = x
                        dhb[b][r, sl] = _sigmoid(x)
                        out[g] = out[g] + x
                        out[grp + g] = out[grp + g] + x * x
                    return tuple(out)

                local0 = tuple(jnp.zeros((L,), jnp.float32)
                               for _ in range(2 * grp))
                local = lax.fori_loop(0, CHUNK, row_body, local0, unroll=False)
                mask = jnp.where(live, 1.0, 0.0)
                bn = tuple(bn[i] + local[i] * mask for i in range(2 * grp))

                @pl.when(live)
                def _():
                    out_issue(k, b)
            return bn

        bn = lax.fori_loop(0, kmax // 2, outer_body, carry0, unroll=False)

        for g in range(grp):
            bns_b[pl.ds(g * L, L)] = bn[g]
            bnq_b[pl.ds(g * L, L)] = bn[grp + g]
        pltpu.sync_copy(bns_b, bns_o.at[pl.ds(wid * d, d)])
        pltpu.sync_copy(bnq_b, bnq_o.at[pl.ds(wid * d, d)])

        plsc.subcore_barrier()

        @pl.when(s < NS - 1)
        def _():
            _spmem_dump(acc, dh0, psum_o, c * n, row0, rps)

        @pl.when(s == NS - 1)
        def _():
            _spmem_dump(acc, dh0, psum_o, c * n, row0, rps_last)

    return pass1


def _make_pass2(n, e_cnt, d):
    nch = e_cnt // CHUNK
    base_chunks = nch // NW
    extra = nch % NW
    rps, rps_last = _row_split(n)
    grp = d // L
    mesh = plsc.VectorSubcoreMesh(core_axis_name="c", subcore_axis_name="s",
                                  num_cores=NC, num_subcores=NS)

    kmax = base_chunks + (1 if extra else 0) + 1
    if kmax % 2:
        kmax += 1

    @functools.partial(
        pl.kernel,
        out_type=[
            jax.ShapeDtypeStruct((NC * n, d), jnp.float32),
        ],
        mesh=mesh,
        scratch_types=[
            pltpu.VMEM_SHARED((n, d), jnp.float32),
            pltpu.VMEM((CHUNK,), jnp.int32),
            pltpu.VMEM((CHUNK,), jnp.int32),
            pltpu.VMEM((CHUNK,), jnp.int32),
            pltpu.VMEM((CHUNK,), jnp.int32),
            pltpu.VMEM((CHUNK,), jnp.int32),
            pltpu.VMEM((CHUNK,), jnp.int32),
            pltpu.VMEM((CHUNK, d), jnp.float32),
            pltpu.VMEM((CHUNK, d), jnp.float32),
            pltpu.VMEM((CHUNK, d), jnp.float32),
            pltpu.VMEM((CHUNK, d), jnp.float32),
            pltpu.SemaphoreType.DMA((2,)),
            pltpu.SemaphoreType.DMA((2,)),
            pltpu.SemaphoreType.DMA((2,)),
            pltpu.SemaphoreType.DMA((2,)),
            pltpu.SemaphoreType.DMA((2,)),
        ],
    )
    def pass2(enew, eee, src, dst, psum_o,
              acc, is0, is1, id0, id1, ic0, ic1, en0, en1, g0, g1,
              sis, sid, sen, sg, sos):
        c = lax.axis_index("c")
        s = lax.axis_index("s")
        wid = s * NC + c
        isb, idb, icb = (is0, is1), (id0, id1), (ic0, ic1)
        enb, gb = (en0, en1), (g0, g1)

        _zero_fill(g0, CHUNK)
        row0 = s * rps

        @pl.when(s < NS - 1)
        def _():
            _spmem_zero(acc, g0, row0, rps)

        @pl.when(s == NS - 1)
        def _():
            _spmem_zero(acc, g0, row0, rps_last)

        plsc.subcore_barrier()

        n_my = base_chunks + jnp.where(wid < extra, 1, 0)

        def idx_issue(k, b):
            base = (k * NW + wid) * CHUNK
            pltpu.async_copy(src.at[pl.ds(base, CHUNK)], isb[b], sis.at[b])
            pltpu.async_copy(dst.at[pl.ds(base, CHUNK)], idb[b], sid.at[b])

        def idx_wait(b):
            pltpu.make_async_copy(src.at[pl.ds(0, CHUNK)], isb[b],
                                  sis.at[b]).wait()
            pltpu.make_async_copy(dst.at[pl.ds(0, CHUNK)], idb[b],
                                  sid.at[b]).wait()

        def in_issue(k, b):
            base = (k * NW + wid) * CHUNK
            pltpu.async_copy(enew.at[pl.ds(base, CHUNK)], enb[b], sen.at[b])
            pltpu.async_copy(eee.at[isb[b]], gb[b], sg.at[b])

        def in_wait(b):
            pltpu.make_async_copy(enew.at[pl.ds(0, CHUNK)], enb[b],
                                  sen.at[b]).wait()
            pltpu.make_async_copy(eee.at[isb[b]], gb[b], sg.at[b]).wait()

        def out_issue(b):
            pltpu.async_copy(gb[b], acc.at[icb[b]], sos.at[b], add=True)

        def out_wait(b):
            pltpu.make_async_copy(gb[b], acc.at[icb[b]], sos.at[b]).wait()

        def idx_keep(b):
            for g in range(CHUNK // L):
                icb[b][pl.ds(g * L, L)] = idb[b][pl.ds(g * L, L)]

        idx_issue(0, 0)
        idx_issue(1, 1)
        idx_wait(0)
        in_issue(0, 0)

        def outer_body(kk, carry):
            for b in (0, 1):
                k = 2 * kk + b
                nb = 1 - b
                live = k < n_my

                @pl.when(live)
                def _():
                    in_wait(b)
                    idx_keep(b)

                @pl.when(jnp.logical_and(k >= 1, k - 1 < n_my))
                def _():
                    out_wait(nb)

                @pl.when(k + 1 < n_my)
                def _():
                    idx_wait(nb)
                    in_issue(k + 1, nb)

                @pl.when(k + 2 < n_my)
                def _():
                    idx_issue(k + 2, b)

                def row_body(r, rc):
                    for g in range(grp):
                        sl = pl.ds(g * L, L)
                        sg_v = _sigmoid(enb[b][r, sl])
                        gb[b][r, sl] = gb[b][r, sl] * sg_v
                    return rc

                lax.fori_loop(0, CHUNK, row_body, 0, unroll=False)

                @pl.when(live)
                def _():
                    out_issue(b)
            return carry

        lax.fori_loop(0, kmax // 2, outer_body, 0, unroll=False)

        plsc.subcore_barrier()

        @pl.when(s < NS - 1)
        def _():
            _spmem_dump(acc, g0, psum_o, c * n, row0, rps)

        @pl.when(s == NS - 1)
        def _():
            _spmem_dump(acc, g0, psum_o, c * n, row0, rps_last)

    return pass2


def kernel(h, e, edge_index, A_w, A_b, B_w, B_b, C_w, C_b, D_w, D_b, E_w, E_b,
           bn_h_gamma, bn_h_beta, bn_e_gamma, bn_e_beta):
    n, d = h.shape
    e_cnt = e.shape[0]
    src = edge_index[0].astype(jnp.int32)
    dst = edge_index[1].astype(jnp.int32)

    dh, eh = pl.pallas_call(
        _node_mm2_body,
        out_shape=[jax.ShapeDtypeStruct((n, d), jnp.float32)] * 2,
    )(h, D_w, D_b.reshape(1, d), E_w, E_b.reshape(1, d))
    ah, bh = pl.pallas_call(
        _node_mm2_body,
        out_shape=[jax.ShapeDtypeStruct((n, d), jnp.float32)] * 2,
    )(h, A_w, A_b.reshape(1, d), B_w, B_b.reshape(1, d))

    br = 4000
    ce = pl.pallas_call(
        _edge_mm_body,
        grid=(e_cnt // br,),
        in_specs=[
            pl.BlockSpec((br, d), lambda i: (i, 0)),
            pl.BlockSpec((d, d), lambda i: (0, 0)),
            pl.BlockSpec((1, d), lambda i: (0, 0)),
        ],
        out_specs=pl.BlockSpec((br, d), lambda i: (i, 0)),
        out_shape=jax.ShapeDtypeStruct((e_cnt, d), jnp.float32),
    )(e, C_w, C_b.reshape(1, d))

    enew, psum, bns, bnq = _make_pass1(n, e_cnt, d)(ce, dh, eh, src, dst)

    eee, esc, esh = pl.pallas_call(
        functools.partial(_stage_b_body, n_edges=float(e_cnt)),
        out_shape=[
            jax.ShapeDtypeStruct((n, d), jnp.float32),
            jax.ShapeDtypeStruct((8, d), jnp.float32),
            jax.ShapeDtypeStruct((8, d), jnp.float32),
        ],
    )(psum, bh, bns.reshape(NW, d), bnq.reshape(NW, d),
      bn_e_gamma.reshape(1, d), bn_e_beta.reshape(1, d))

    br2 = 4000
    e_out = pl.pallas_call(
        _eout_body,
        grid=(e_cnt // br2,),
        in_specs=[
            pl.BlockSpec((br2, d), lambda i: (i, 0)),
            pl.BlockSpec((br2, d), lambda i: (i, 0)),
            pl.BlockSpec((1, d), lambda i: (0, 0)),
            pl.BlockSpec((1, d), lambda i: (0, 0)),
        ],
        out_specs=pl.BlockSpec((br2, d), lambda i: (i, 0)),
        out_shape=jax.ShapeDtypeStruct((e_cnt, d), jnp.float32),
    )(enew, e, esc[0].reshape(1, d), esh[0].reshape(1, d))

    (psumh,) = _make_pass2(n, e_cnt, d)(enew, eee, src, dst)

    h_out = pl.pallas_call(
        _stage_c_body,
        out_shape=jax.ShapeDtypeStruct((n, d), jnp.float32),
    )(ah, psumh, h, bn_h_gamma.reshape(1, d), bn_h_beta.reshape(1, d))

    return (h_out, e_out)
